# baseline (speedup 1.0000x reference)
import jax
import jax.numpy as jnp
from jax.experimental import pallas as pl
from jax.experimental.pallas import tpu as pltpu

B, N, F, S = 8, 64, 128, 16
F_ = 256
HID = 256
HB = 2
HC = HID // HB
PACK = 128 // S


def _ecc_kernel(x_ref, adj_ref, adj2_ref, e_ref, w1_ref, b1_ref, w2_ref,
                b2_ref, root_ref, bias_ref, out_ref, k_ref, acc_ref):
    hb = pl.program_id(0)
    a = pl.program_id(1)

    @pl.when(a == 0)
    def _build_k():
        k = jax.lax.dot_general(
            x_ref[...].astype(jnp.bfloat16), w2_ref[...],
            (((1,), (1,)), ((), ())), preferred_element_type=jnp.float32)
        k_ref[...] = k.astype(jnp.bfloat16)

    h1 = jnp.dot(e_ref[0].astype(jnp.bfloat16), w1_ref[0],
                 preferred_element_type=jnp.float32)
    h1 = h1.reshape(N * N, HC) + b1_ref[0]
    g = jnp.maximum(h1, 0.0) * adj_ref[0]
    g2 = g.astype(jnp.bfloat16).reshape(N, N * HC)

    k2 = k_ref[pl.ds(a * N, N), :].reshape(N * HC, F_)
    contrib = jnp.dot(g2, k2, preferred_element_type=jnp.float32)

    @pl.when(hb == 0)
    def _init_acc():
        acc_ref[a] = contrib

    @pl.when(hb == HB - 1)
    def _finish():
        x_a = x_ref[pl.ds(a * N, N), :]
        bx = jax.lax.dot_general(
            x_a, b2_ref[...], (((1,), (1,)), ((), ())),
            preferred_element_type=jnp.float32)
        deg = jnp.dot(adj2_ref[0], bx, preferred_element_type=jnp.float32)
        rt = jnp.dot(x_a, root_ref[...], preferred_element_type=jnp.float32)
        out_ref[0] = jnp.maximum(acc_ref[a] + contrib + deg + rt
                                 + bias_ref[...], 0.0)


def kernel(x, adj, e, W1, b1, W2, b2, root, bias):
    x_all = x.reshape(B * N, F)
    adj3 = adj.reshape(B, N * N, 1)
    e_pack = e.reshape(B, N * N // PACK, S * PACK)
    eye = jnp.eye(PACK, dtype=W1.dtype)
    w1t = jnp.stack([jnp.kron(eye, W1[:, i * HC:(i + 1) * HC])
                     for i in range(HB)])
    b1c = b1.reshape(HB, 1, HC)
    w2cf = W2.reshape(HID * F_, F).astype(jnp.bfloat16)
    b2m = b2.reshape(F_, F)
    bias2 = bias.reshape(1, F_)
    out = pl.pallas_call(
        _ecc_kernel,
        grid=(HB, B),
        in_specs=[
            pl.BlockSpec((B * N, F), lambda hb, a: (0, 0)),
            pl.BlockSpec((1, N * N, 1), lambda hb, a: (a, 0, 0)),
            pl.BlockSpec((1, N, N), lambda hb, a: (a, 0, 0)),
            pl.BlockSpec((1, N * N // PACK, S * PACK),
                         lambda hb, a: (a, 0, 0)),
            pl.BlockSpec((1, S * PACK, PACK * HC),
                         lambda hb, a: (hb, 0, 0)),
            pl.BlockSpec((1, 1, HC), lambda hb, a: (hb, 0, 0)),
            pl.BlockSpec((HC * F_, F), lambda hb, a: (hb, 0)),
            pl.BlockSpec((F_, F), lambda hb, a: (0, 0)),
            pl.BlockSpec((F, F_), lambda hb, a: (0, 0)),
            pl.BlockSpec((1, F_), lambda hb, a: (0, 0)),
        ],
        out_specs=pl.BlockSpec((1, N, F_), lambda hb, a: (a, 0, 0)),
        out_shape=jax.ShapeDtypeStruct((B, N, F_), jnp.float32),
        scratch_shapes=[
            pltpu.VMEM((B * N, HC * F_), jnp.bfloat16),
            pltpu.VMEM((B, N, F_), jnp.float32),
        ],
        compiler_params=pltpu.CompilerParams(
            dimension_semantics=("arbitrary", "arbitrary"),
            vmem_limit_bytes=64 * 1024 * 1024),
    )(x_all, adj3, adj, e_pack, w1t, b1c, w2cf, b2m, root, bias2)
    return out
